# baseline (speedup 1.0000x reference)
import functools

import jax
import jax.numpy as jnp
from jax import lax
from jax.experimental import pallas as pl
from jax.experimental.pallas import tpu as pltpu
from jax.experimental.pallas import tpu_sc as plsc

B, L, V, D = 4096, 50, 100000, 64
CUM = [0, 2, 10, 21]
LABEL = 21

NC, NS = 2, 16
NW = NC * NS
CB = 2
PER_W = B // NW
CHUNKS = PER_W // CB
NBUF = 4
ROW = 112
USED = 104

def _pattern_vec(j):
    d = lax.iota(jnp.int32, 16) + 16 * j
    return jnp.where(d < 50, d,
                     jnp.where(d < 102, d - 2,
                               jnp.where(d < 104, d - 4, d - 16)))


RB = 2000


def _tc_linearize(table):
    def body(a_ref, b_ref, o_ref):
        o_ref[:, 0:D] = a_ref[...]
        o_ref[:, D:2 * D] = b_ref[...]

    nblk = (V // 2) // RB
    return pl.pallas_call(
        body,
        grid=(nblk,),
        in_specs=[
            pl.BlockSpec((RB, D), lambda i: (i, 0)),
            pl.BlockSpec((RB, D), lambda i: (i + nblk, 0)),
        ],
        out_specs=pl.BlockSpec((RB, 2 * D), lambda i: (i, 0)),
        out_shape=jax.ShapeDtypeStruct((V // 2, 2 * D), jnp.float32),
    )(table, table)


def _sc_pool(tpair, x_flat):
    mesh = plsc.VectorSubcoreMesh(core_axis_name="c", subcore_axis_name="s")

    @functools.partial(
        pl.kernel,
        mesh=mesh,
        out_type=jax.ShapeDtypeStruct((B, D), jnp.float32),
        scratch_types=[
            pltpu.VMEM((PER_W * L,), jnp.int32),
            pltpu.VMEM((CHUNKS, ROW), jnp.int32),
            pltpu.VMEM((CHUNKS * ROW,), jnp.int32),
            pltpu.VMEM((USED, 2 * D), jnp.float32),
            pltpu.VMEM((USED, 2 * D), jnp.float32),
            pltpu.VMEM((USED, 2 * D), jnp.float32),
            pltpu.VMEM((USED, 2 * D), jnp.float32),
            pltpu.VMEM((PER_W, D), jnp.float32),
            pltpu.SemaphoreType.DMA,
            pltpu.SemaphoreType.DMA,
            pltpu.SemaphoreType.DMA,
            pltpu.SemaphoreType.DMA,
        ],
        compiler_params=pltpu.CompilerParams(use_tc_tiling_on_sc=False, needs_layout_passes=False),
    )
    def pool(tab_hbm, x_hbm, out_hbm,
             xloc, idx_v, col_v, buf0, buf1, buf2, buf3, acc,
             sem0, sem1, sem2, sem3):
        bufs = (buf0, buf1, buf2, buf3)
        sems = (sem0, sem1, sem2, sem3)
        wid = lax.axis_index("s") * NC + lax.axis_index("c")
        base = wid * PER_W

        pltpu.sync_copy(x_hbm.at[pl.ds(base * L, PER_W * L)], xloc)

        pats = [_pattern_vec(j) for j in range(ROW // 16)]

        def build(i, carry):
            cb = i * (CB * L)
            for j in range(ROW // 16):
                v = plsc.load_gather(xloc, [pats[j] + cb])
                hi = v >= (V // 2)
                idx_v[i, pl.ds(16 * j, 16)] = jnp.where(hi, v - V // 2, v)
                col_v[pl.ds(i * ROW + 16 * j, 16)] = jnp.where(hi, D, 0)
            return carry

        lax.fori_loop(0, CHUNKS, build, 0)

        def start(i, b):
            pltpu.async_copy(
                tab_hbm.at[idx_v.at[i, pl.ds(0, USED)]], bufs[b], sems[b])

        lanes = lax.iota(jnp.int32, 16)

        def seg_sum(buf, i, r0):
            cbase = i * ROW + r0

            def body(r, carry):
                cb = plsc.load_gather(col_v, [jnp.broadcast_to(cbase + r, (16,))])
                rowv = jnp.broadcast_to(r0 + r, (16,))
                cols = cb + lanes
                new = []
                for q in range(4):
                    val = plsc.load_gather(buf, [rowv, cols + 16 * q])
                    new.append(carry[q] + val)
                return tuple(new)
            z = jnp.zeros((16,), jnp.float32)
            return lax.fori_loop(0, L, body, (z, z, z, z), unroll=10)

        for b in range(NBUF):
            start(b, b)

        def outer(j, carry):
            for b in range(NBUF):
                i = j * NBUF + b
                pltpu.make_async_copy(
                    tab_hbm.at[idx_v.at[i, pl.ds(0, USED)]],
                    bufs[b], sems[b]).wait()
                for s2 in range(CB):
                    a = seg_sum(bufs[b], i, s2 * 52)
                    row_l = CB * i + s2
                    for q in range(4):
                        acc[row_l, pl.ds(16 * q, 16)] = a[q]

                @pl.when(i + NBUF < CHUNKS)
                def _():
                    start(i + NBUF, b)
            return carry

        lax.fori_loop(0, CHUNKS // NBUF, outer, 0)
        pltpu.sync_copy(acc, out_hbm.at[pl.ds(base, PER_W)])

    return pool(tpair, x_flat)


def _tc_heads(user_sum, maskf, y, ob, wcat, bcat):
    def body(us_ref, mask_ref, y_ref, ob_ref, w_ref, b_ref,
             logit_ref, loss_ref):
        xlen = jnp.sum(mask_ref[...].astype(jnp.float32), axis=1,
                       keepdims=True)
        ur = us_ref[...] / xlen
        lg = jnp.dot(ur, w_ref[...], preferred_element_type=jnp.float32)
        wc = (lg + b_ref[...]) * ob_ref[...]
        logit_ref[...] = wc
        total = jnp.float32(0.0)
        for i in range(3):
            s, e = CUM[i], CUM[i + 1]
            sl = wc[:, s:e]
            m = jnp.max(sl, axis=1, keepdims=True)
            lse = jnp.log(jnp.sum(jnp.exp(sl - m), axis=1, keepdims=True)) + m
            logp = sl - lse
            total = total - jnp.sum(y_ref[:, s:e] * logp) / B
        loss_ref[...] = jnp.reshape(total, (1, 1))

    return pl.pallas_call(
        body,
        out_shape=[
            jax.ShapeDtypeStruct((B, LABEL), jnp.float32),
            jax.ShapeDtypeStruct((1, 1), jnp.float32),
        ],
    )(user_sum, maskf, y, ob, wcat, bcat)


def kernel(x, x_mask, y, ob, table, W0, b0, W1, b1, W2, b2):
    x_flat = x.astype(jnp.int32).reshape(B * L)
    user_sum = _sc_pool(_tc_linearize(table), x_flat)
    wcat = jnp.concatenate([W0, W1, W2], axis=1)
    bcat = jnp.concatenate([b0, b1, b2]).reshape(1, LABEL)
    logit, loss2d = _tc_heads(user_sum, x_mask, y, ob, wcat, bcat)
    return logit, loss2d[0, 0]


# speedup vs baseline: 1.1317x; 1.1317x over previous
import functools

import jax
import jax.numpy as jnp
from jax import lax
from jax.experimental import pallas as pl
from jax.experimental.pallas import tpu as pltpu
from jax.experimental.pallas import tpu_sc as plsc

B, L, V, D = 4096, 50, 100000, 64
CUM = [0, 2, 10, 21]
LABEL = 21

NC, NS = 2, 16
NW = NC * NS
CB = 2
PER_W = B // NW
CHUNKS = PER_W // CB
NBUF = 4
ROW = 112
USED = 104

def _pattern_vec(j):
    d = lax.iota(jnp.int32, 16) + 16 * j
    return jnp.where(d < 50, d,
                     jnp.where(d < 102, d - 2,
                               jnp.where(d < 104, d - 4, d - 16)))


RB = 2000


def _tc_pack(table):
    def body(a_ref, b_ref, o_ref):
        o_ref[:, 0:D] = a_ref[...]
        o_ref[:, D:2 * D] = b_ref[...]

    nblk = (V // 2) // RB
    return pl.pallas_call(
        body,
        grid=(nblk,),
        in_specs=[
            pl.BlockSpec((RB, D), lambda i: (i, 0)),
            pl.BlockSpec((RB, D), lambda i: (i + nblk, 0)),
        ],
        out_specs=pl.BlockSpec((RB, 2 * D), lambda i: (i, 0)),
        out_shape=jax.ShapeDtypeStruct((V // 2, 2 * D), jnp.float32),
    )(table, table)


def _sc_pool(table, x_flat):
    mesh = plsc.VectorSubcoreMesh(core_axis_name="c", subcore_axis_name="s")

    @functools.partial(
        pl.kernel,
        mesh=mesh,
        out_type=jax.ShapeDtypeStruct((B, D), jnp.float32),
        scratch_types=[
            pltpu.VMEM((PER_W * L,), jnp.int32),
            pltpu.VMEM((CHUNKS, ROW), jnp.int32),
            pltpu.VMEM((USED, D), jnp.float32),
            pltpu.VMEM((USED, D), jnp.float32),
            pltpu.VMEM((USED, D), jnp.float32),
            pltpu.VMEM((USED, D), jnp.float32),
            pltpu.VMEM((PER_W, D), jnp.float32),
            pltpu.SemaphoreType.DMA,
            pltpu.SemaphoreType.DMA,
            pltpu.SemaphoreType.DMA,
            pltpu.SemaphoreType.DMA,
        ],
        compiler_params=pltpu.CompilerParams(use_tc_tiling_on_sc=False, needs_layout_passes=False),
    )
    def pool(table_hbm, x_hbm, out_hbm,
             xloc, idx_v, buf0, buf1, buf2, buf3, acc,
             sem0, sem1, sem2, sem3):
        bufs = (buf0, buf1, buf2, buf3)
        sems = (sem0, sem1, sem2, sem3)
        wid = lax.axis_index("s") * NC + lax.axis_index("c")
        base = wid * PER_W

        pltpu.sync_copy(x_hbm.at[pl.ds(base * L, PER_W * L)], xloc)

        pats = [_pattern_vec(j) for j in range(ROW // 16)]

        def build(i, carry):
            cb = i * (CB * L)
            for j in range(ROW // 16):
                v = plsc.load_gather(xloc, [pats[j] + cb])
                idx_v[i, pl.ds(16 * j, 16)] = jnp.where(
                    v < V // 2, 2 * v, 2 * v - (V - 1))
            return carry

        lax.fori_loop(0, CHUNKS, build, 0)

        def start(i, b):
            pltpu.async_copy(
                table_hbm.at[idx_v.at[i, pl.ds(0, USED)]], bufs[b], sems[b])

        def seg_sum(buf, r0):
            def body(k, carry):
                row = r0 + 2 * k
                new = []
                for q in range(4):
                    new.append(carry[q] + buf[row, pl.ds(16 * q, 16)])
                for q in range(4):
                    new.append(carry[4 + q] + buf[row + 1, pl.ds(16 * q, 16)])
                return tuple(new)
            z = jnp.zeros((16,), jnp.float32)
            r = lax.fori_loop(0, L // 2, body, (z,) * 8, unroll=5)
            return tuple(r[q] + r[4 + q] for q in range(4))

        for b in range(NBUF):
            start(b, b)

        def outer(j, carry):
            for b in range(NBUF):
                i = j * NBUF + b
                pltpu.make_async_copy(
                    table_hbm.at[idx_v.at[i, pl.ds(0, USED)]],
                    bufs[b], sems[b]).wait()
                for s2 in range(CB):
                    a = seg_sum(bufs[b], s2 * 52)
                    row_l = CB * i + s2
                    for q in range(4):
                        acc[row_l, pl.ds(16 * q, 16)] = a[q]

                @pl.when(i + NBUF < CHUNKS)
                def _():
                    start(i + NBUF, b)
            return carry

        lax.fori_loop(0, CHUNKS // NBUF, outer, 0)
        pltpu.sync_copy(acc, out_hbm.at[pl.ds(base, PER_W)])

    return pool(table, x_flat)


def _tc_heads(user_sum, maskf, y, ob, wcat, bcat):
    def body(us_ref, mask_ref, y_ref, ob_ref, w_ref, b_ref,
             logit_ref, loss_ref):
        xlen = jnp.sum(mask_ref[...].astype(jnp.float32), axis=1,
                       keepdims=True)
        ur = us_ref[...] / xlen
        lg = jnp.dot(ur, w_ref[...], preferred_element_type=jnp.float32)
        wc = (lg + b_ref[...]) * ob_ref[...]
        logit_ref[...] = wc
        total = jnp.float32(0.0)
        for i in range(3):
            s, e = CUM[i], CUM[i + 1]
            sl = wc[:, s:e]
            m = jnp.max(sl, axis=1, keepdims=True)
            lse = jnp.log(jnp.sum(jnp.exp(sl - m), axis=1, keepdims=True)) + m
            logp = sl - lse
            total = total - jnp.sum(y_ref[:, s:e] * logp) / B
        loss_ref[...] = jnp.reshape(total, (1, 1))

    return pl.pallas_call(
        body,
        out_shape=[
            jax.ShapeDtypeStruct((B, LABEL), jnp.float32),
            jax.ShapeDtypeStruct((1, 1), jnp.float32),
        ],
    )(user_sum, maskf, y, ob, wcat, bcat)


def kernel(x, x_mask, y, ob, table, W0, b0, W1, b1, W2, b2):
    x_flat = x.astype(jnp.int32).reshape(B * L)
    user_sum = _sc_pool(_tc_pack(table).reshape(V, D), x_flat)
    wcat = jnp.concatenate([W0, W1, W2], axis=1)
    bcat = jnp.concatenate([b0, b1, b2]).reshape(1, LABEL)
    logit, loss2d = _tc_heads(user_sum, x_mask, y, ob, wcat, bcat)
    return logit, loss2d[0, 0]


# speedup vs baseline: 1.2339x; 1.0903x over previous
import functools

import jax
import jax.numpy as jnp
from jax import lax
from jax.experimental import pallas as pl
from jax.experimental.pallas import tpu as pltpu
from jax.experimental.pallas import tpu_sc as plsc

B, L, V, D = 4096, 50, 100000, 64
CUM = [0, 2, 10, 21]
LABEL = 21

NC, NS = 2, 16
NW = NC * NS
LPAD = 52
CB = 2
PER_W = B // NW
CHUNKS = PER_W // CB
NBUF = 4
CHUNK_IDX = CB * LPAD


def _sc_pool(table, xpad):
    mesh = plsc.VectorSubcoreMesh(core_axis_name="c", subcore_axis_name="s")

    @functools.partial(
        pl.kernel,
        mesh=mesh,
        out_type=jax.ShapeDtypeStruct((B, D), jnp.float32),
        scratch_types=[
            pltpu.VMEM((CHUNKS, CHUNK_IDX), jnp.int32),
            pltpu.VMEM((CHUNK_IDX, D), jnp.float32),
            pltpu.VMEM((CHUNK_IDX, D), jnp.float32),
            pltpu.VMEM((CHUNK_IDX, D), jnp.float32),
            pltpu.VMEM((CHUNK_IDX, D), jnp.float32),
            pltpu.VMEM((PER_W, D), jnp.float32),
            pltpu.SemaphoreType.DMA,
            pltpu.SemaphoreType.DMA,
            pltpu.SemaphoreType.DMA,
            pltpu.SemaphoreType.DMA,
        ],
        compiler_params=pltpu.CompilerParams(use_tc_tiling_on_sc=False),
    )
    def pool(table_hbm, xpad_hbm, out_hbm,
             idx_v, buf0, buf1, buf2, buf3, acc,
             sem0, sem1, sem2, sem3):
        bufs = (buf0, buf1, buf2, buf3)
        sems = (sem0, sem1, sem2, sem3)
        wid = lax.axis_index("s") * NC + lax.axis_index("c")
        base = wid * PER_W

        pltpu.sync_copy(xpad_hbm.at[pl.ds(wid * CHUNKS, CHUNKS)], idx_v)

        def start(i, b):
            pltpu.async_copy(table_hbm.at[idx_v.at[i]], bufs[b], sems[b])

        def seg_sum(buf, r0):
            def body(r, carry):
                row = r0 + r
                return tuple(carry[q] + buf[row, pl.ds(16 * q, 16)]
                             for q in range(4))
            z = jnp.zeros((16,), jnp.float32)
            return lax.fori_loop(0, L, body, (z, z, z, z), unroll=10)

        for b in range(NBUF):
            start(b, b)

        def outer(j, carry):
            for b in range(NBUF):
                i = j * NBUF + b
                pltpu.make_async_copy(
                    table_hbm.at[idx_v.at[i]], bufs[b], sems[b]).wait()
                for s2 in range(CB):
                    a = seg_sum(bufs[b], s2 * LPAD)
                    row_l = CB * i + s2
                    for q in range(4):
                        acc[row_l, pl.ds(16 * q, 16)] = a[q]

                @pl.when(i + NBUF < CHUNKS)
                def _():
                    start(i + NBUF, b)
            return carry

        lax.fori_loop(0, CHUNKS // NBUF, outer, 0)
        pltpu.sync_copy(acc, out_hbm.at[pl.ds(base, PER_W)])

    return pool(table, xpad)


def _tc_heads(user_sum, mask, y, ob, wcat, bcat):
    def body(us_ref, mask_ref, y_ref, ob_ref, w_ref, b_ref,
             logit_ref, loss_ref):
        xlen = jnp.sum(mask_ref[...].astype(jnp.float32), axis=1,
                       keepdims=True)
        ur = us_ref[...] / xlen
        lg = jnp.dot(ur, w_ref[...], preferred_element_type=jnp.float32)
        wc = (lg + b_ref[...]) * ob_ref[...]
        logit_ref[...] = wc
        total = jnp.float32(0.0)
        for i in range(3):
            s, e = CUM[i], CUM[i + 1]
            sl = wc[:, s:e]
            m = jnp.max(sl, axis=1, keepdims=True)
            lse = jnp.log(jnp.sum(jnp.exp(sl - m), axis=1, keepdims=True)) + m
            logp = sl - lse
            total = total - jnp.sum(y_ref[:, s:e] * logp) / B
        loss_ref[...] = jnp.reshape(total, (1, 1))

    return pl.pallas_call(
        body,
        out_shape=[
            jax.ShapeDtypeStruct((B, LABEL), jnp.float32),
            jax.ShapeDtypeStruct((1, 1), jnp.float32),
        ],
    )(user_sum, mask, y, ob, wcat, bcat)


def kernel(x, x_mask, y, ob, table, W0, b0, W1, b1, W2, b2):
    xi = x.astype(jnp.int32)
    xpad = jnp.concatenate([xi, xi[:, :LPAD - L]], axis=1)
    xpad = xpad.reshape(B // CB, CHUNK_IDX)
    user_sum = _sc_pool(table, xpad)
    wcat = jnp.concatenate([W0, W1, W2], axis=1)
    bcat = jnp.concatenate([b0, b1, b2]).reshape(1, LABEL)
    logit, loss2d = _tc_heads(user_sum, x_mask, y, ob, wcat, bcat)
    return logit, loss2d[0, 0]
